# Initial kernel scaffold; baseline (speedup 1.0000x reference)
import jax
import jax.numpy as jnp
from jax.experimental import pallas as pl


def kernel(x):
    raise NotImplementedError("write your pallas kernel here")


# speedup vs baseline: 6.7963x; 6.7963x over previous
import functools

import jax
import jax.numpy as jnp
from jax.experimental import pallas as pl
from jax.experimental.pallas import tpu as pltpu

WL = 16
LOSS_THR = 1.0
BIT = 8


def _delta(xb, imgs, wl, w):
    xl = jnp.roll(xb, 1, axis=2)
    lane = jax.lax.broadcasted_iota(jnp.int32, (imgs, wl, w), 2)
    xl = jnp.where(lane == 0, 0.0, xl)
    return xl, xb - xl


def _absmax_kernel(x_ref, o_ref):
    imgs, wl, w = x_ref.shape
    xb = x_ref[...]
    _, xc = _delta(xb, imgs, wl, w)
    m = jnp.max(jnp.abs(xc))
    o_ref[...] = jnp.full(o_ref.shape, m, jnp.float32)


def _fit_kernel(lsb_ref, x_ref, o_ref):
    imgs, wl, w = x_ref.shape
    wp = w // wl
    lsb = lsb_ref[0]
    inv_lsb = 1.0 / lsb

    xb = x_ref[...]
    xl, xc = _delta(xb, imgs, wl, w)
    x1 = jnp.round(xc * inv_lsb) * lsb

    a1 = x1[0]
    a2 = x1[1]

    colg = jax.lax.broadcasted_iota(jnp.int32, (w, wp), 0) // wl
    segi = jax.lax.broadcasted_iota(jnp.int32, (w, wp), 1)
    seg = (colg == segi).astype(jnp.float32)
    seg_t = (segi.T == colg.T).astype(jnp.float32)

    dot = functools.partial(
        jnp.matmul,
        precision=jax.lax.Precision.HIGHEST,
        preferred_element_type=jnp.float32,
    )

    def wsum2(v):
        return dot(jnp.sum(v, axis=0, keepdims=True), seg)

    s11 = wsum2(a1 * a1)
    s12 = wsum2(a1 * a2)
    s22 = wsum2(a2 * a2)
    s1 = wsum2(a1)
    s2 = wsum2(a2)
    n = jnp.float32(wl * wl)

    def wsum3(v):
        return dot(jnp.sum(v, axis=1), seg)

    t1 = wsum3(x1 * a1[None])
    t2 = wsum3(x1 * a2[None])
    t3 = wsum3(x1)

    m11 = s22 * n - s2 * s2
    m12 = s1 * s2 - s12 * n
    m13 = s12 * s2 - s1 * s22
    det = s11 * m11 + s12 * m12 + s1 * m13
    m22 = s11 * n - s1 * s1
    m23 = s12 * s1 - s11 * s2
    m33 = s11 * s22 - s12 * s12
    det0 = det == 0.0
    rdet = 1.0 / jnp.where(det0, 1.0, det)
    i11 = jnp.where(det0, 1.0, m11 * rdet)
    i12 = jnp.where(det0, 0.0, m12 * rdet)
    i13 = jnp.where(det0, 0.0, m13 * rdet)
    i22 = jnp.where(det0, 1.0, m22 * rdet)
    i23 = jnp.where(det0, 0.0, m23 * rdet)
    i33 = jnp.where(det0, 1.0, m33 * rdet)

    c1 = i11 * t1 + i12 * t2 + i13 * t3
    c2 = i12 * t1 + i22 * t2 + i23 * t3
    c3 = i13 * t1 + i23 * t2 + i33 * t3

    c1e = dot(c1, seg_t)[:, None, :]
    c2e = dot(c2, seg_t)[:, None, :]
    c3e = dot(c3, seg_t)[:, None, :]

    r = c1e * a1[None] + c2e * a2[None] + c3e
    r1 = jnp.round(r * inv_lsb) * lsb

    diff = x1 - r1
    loss = wsum3(diff * diff)
    keep = (loss <= LOSS_THR).astype(jnp.float32)
    keep_e = dot(keep, seg_t)[:, None, :]

    sel = jnp.where(keep_e > 0.5, r1, x1)
    o_ref[...] = sel + xl


def kernel(x):
    _, imgs, h, w = x.shape
    hp = h // WL
    x3 = x[0]

    maxes = pl.pallas_call(
        _absmax_kernel,
        grid=(hp,),
        in_specs=[pl.BlockSpec((imgs, WL, w), lambda i: (0, i, 0))],
        out_specs=pl.BlockSpec((1, 8, 128), lambda i: (i, 0, 0)),
        out_shape=jax.ShapeDtypeStruct((hp, 8, 128), jnp.float32),
    )(x3)

    in_max = jnp.max(maxes)
    lsb = 2.0 ** (jnp.round(jnp.log2(in_max / 2.0 ** (BIT - 1))) + 1.0)
    lsb_arr = lsb.reshape(1).astype(jnp.float32)

    out = pl.pallas_call(
        _fit_kernel,
        grid=(hp,),
        in_specs=[
            pl.BlockSpec(memory_space=pltpu.SMEM),
            pl.BlockSpec((imgs, WL, w), lambda i: (0, i, 0)),
        ],
        out_specs=pl.BlockSpec((imgs, WL, w), lambda i: (0, i, 0)),
        out_shape=jax.ShapeDtypeStruct((imgs, h, w), jnp.float32),
    )(lsb_arr, x3)

    return out[None]


# speedup vs baseline: 8.1397x; 1.1977x over previous
import jax
import jax.numpy as jnp
from jax.experimental import pallas as pl
from jax.experimental.pallas import tpu as pltpu

WL = 16
LOSS_THR = 1.0
BIT = 8


def _delta(xb, imgs, wl, w):
    xl = jnp.roll(xb, 1, axis=2)
    lane = jax.lax.broadcasted_iota(jnp.int32, (imgs, wl, w), 2)
    xl = jnp.where(lane == 0, 0.0, xl)
    return xl, xb - xl


def _absmax_kernel(x_ref, o_ref):
    imgs, wl, w = x_ref.shape
    xb = x_ref[...]
    _, xc = _delta(xb, imgs, wl, w)
    m = jnp.max(jnp.abs(xc))
    o_ref[...] = jnp.full(o_ref.shape, m, jnp.float32)


def _fit_kernel(maxes_ref, seg_ref, seg_t_ref, x_ref, o_ref):
    imgs, wl, w = x_ref.shape
    in_max = jnp.max(maxes_ref[...])
    lsb = 2.0 ** (jnp.round(jnp.log2(in_max / 2.0 ** (BIT - 1))) + 1.0)
    inv_lsb = 1.0 / lsb

    xb = x_ref[...]
    xl, xc = _delta(xb, imgs, wl, w)
    u = jnp.round(xc * inv_lsb)

    ua1 = u[0]
    ua2 = u[1]

    seg = seg_ref[...]
    seg_t = seg_t_ref[...]

    def contract(v):
        part = jax.lax.dot_general(
            v, seg, (((2,), (0,)), ((), ())),
            preferred_element_type=jnp.float32)
        return jnp.sum(part, axis=1)

    t1 = contract(u * ua1[None])
    t2 = contract(u * ua2[None])
    t3 = contract(u)

    s11 = t1[0:1]
    s12 = t1[1:2]
    s22 = t2[1:2]
    s1 = t3[0:1]
    s2 = t3[1:2]
    n = jnp.float32(wl * wl)

    m11 = s22 * n - s2 * s2
    m12 = s1 * s2 - s12 * n
    m13 = s12 * s2 - s1 * s22
    det = s11 * m11 + s12 * m12 + s1 * m13
    m22 = s11 * n - s1 * s1
    m23 = s12 * s1 - s11 * s2
    m33 = s11 * s22 - s12 * s12
    det0 = det == 0.0
    rdet = 1.0 / jnp.where(det0, 1.0, det)
    i11 = jnp.where(det0, 1.0, m11 * rdet)
    i12 = jnp.where(det0, 0.0, m12 * rdet)
    i13 = jnp.where(det0, 0.0, m13 * rdet)
    i22 = jnp.where(det0, 1.0, m22 * rdet)
    i23 = jnp.where(det0, 0.0, m23 * rdet)
    i33 = jnp.where(det0, 1.0, m33 * rdet)

    c1 = i11 * t1 + i12 * t2 + i13 * t3
    c2 = i12 * t1 + i22 * t2 + i23 * t3
    c3 = i13 * t1 + i23 * t2 + i33 * t3

    def expand(v):
        return jax.lax.dot_general(
            v, seg_t, (((1,), (0,)), ((), ())),
            preferred_element_type=jnp.float32)[:, None, :]

    r = expand(c1) * ua1[None] + expand(c2) * ua2[None] + expand(c3)
    r1 = jnp.round(r)

    diff = u - r1
    loss = contract(diff * diff)
    keep = (loss * (lsb * lsb) <= LOSS_THR).astype(jnp.float32)
    keep_e = expand(keep)

    sel = u + keep_e * (r1 - u)
    o_ref[...] = sel * lsb + xl


def kernel(x):
    _, imgs, h, w = x.shape
    hp = h // WL
    wp = w // WL
    x3 = x[0]

    maxes = pl.pallas_call(
        _absmax_kernel,
        grid=(hp,),
        in_specs=[pl.BlockSpec((imgs, WL, w), lambda i: (0, i, 0))],
        out_specs=pl.BlockSpec((1, 8, 128), lambda i: (i, 0, 0)),
        out_shape=jax.ShapeDtypeStruct((hp, 8, 128), jnp.float32),
    )(x3)

    cols = jnp.arange(w, dtype=jnp.int32) // WL
    segs = jnp.arange(wp, dtype=jnp.int32)
    seg = (cols[:, None] == segs[None, :]).astype(jnp.float32)
    seg_t = seg.T

    out = pl.pallas_call(
        _fit_kernel,
        grid=(hp,),
        in_specs=[
            pl.BlockSpec((hp, 8, 128), lambda i: (0, 0, 0)),
            pl.BlockSpec((w, wp), lambda i: (0, 0)),
            pl.BlockSpec((wp, w), lambda i: (0, 0)),
            pl.BlockSpec((imgs, WL, w), lambda i: (0, i, 0)),
        ],
        out_specs=pl.BlockSpec((imgs, WL, w), lambda i: (0, i, 0)),
        out_shape=jax.ShapeDtypeStruct((imgs, h, w), jnp.float32),
    )(maxes, seg, seg_t, x3)

    return out[None]


# speedup vs baseline: 8.4878x; 1.0428x over previous
import jax
import jax.numpy as jnp
from jax.experimental import pallas as pl
from jax.experimental.pallas import tpu as pltpu

WL = 16
LOSS_THR = 1.0
BIT = 8


def _delta(xb, imgs, wl, w):
    xl = jnp.roll(xb, 1, axis=2)
    lane = jax.lax.broadcasted_iota(jnp.int32, (imgs, wl, w), 2)
    xl = jnp.where(lane == 0, 0.0, xl)
    return xl, xb - xl


def _fused_kernel(seg_ref, seg_t_ref, x_ref, o_ref, acc_ref):
    imgs, wl, w = x_ref.shape
    hp = pl.num_programs(0) // 2
    i = pl.program_id(0)

    @pl.when(i < hp)
    def _max_phase():
        xb = x_ref[...]
        _, xc = _delta(xb, imgs, wl, w)
        m = jnp.full((8, 128), jnp.max(jnp.abs(xc)), jnp.float32)
        prev = jnp.where(i == 0, jnp.zeros((8, 128), jnp.float32), acc_ref[...])
        acc_ref[...] = jnp.maximum(prev, m)

    @pl.when(i >= hp)
    def _fit_phase():
        in_max = jnp.max(acc_ref[...])
        lsb = 2.0 ** (jnp.round(jnp.log2(in_max / 2.0 ** (BIT - 1))) + 1.0)
        inv_lsb = 1.0 / lsb

        xb = x_ref[...]
        xl, xc = _delta(xb, imgs, wl, w)
        u = jnp.round(xc * inv_lsb)

        ua1 = u[0]
        ua2 = u[1]

        seg = seg_ref[...]
        seg_t = seg_t_ref[...]

        def contract(v):
            part = jax.lax.dot_general(
                v, seg, (((2,), (0,)), ((), ())),
                preferred_element_type=jnp.float32)
            return jnp.sum(part, axis=1)

        t1 = contract(u * ua1[None])
        t2 = contract(u * ua2[None])
        t3 = contract(u)

        s11 = t1[0:1]
        s12 = t1[1:2]
        s22 = t2[1:2]
        s1 = t3[0:1]
        s2 = t3[1:2]
        n = jnp.float32(wl * wl)

        m11 = s22 * n - s2 * s2
        m12 = s1 * s2 - s12 * n
        m13 = s12 * s2 - s1 * s22
        det = s11 * m11 + s12 * m12 + s1 * m13
        m22 = s11 * n - s1 * s1
        m23 = s12 * s1 - s11 * s2
        m33 = s11 * s22 - s12 * s12
        det0 = det == 0.0
        rdet = 1.0 / jnp.where(det0, 1.0, det)
        i11 = jnp.where(det0, 1.0, m11 * rdet)
        i12 = jnp.where(det0, 0.0, m12 * rdet)
        i13 = jnp.where(det0, 0.0, m13 * rdet)
        i22 = jnp.where(det0, 1.0, m22 * rdet)
        i23 = jnp.where(det0, 0.0, m23 * rdet)
        i33 = jnp.where(det0, 1.0, m33 * rdet)

        c1 = i11 * t1 + i12 * t2 + i13 * t3
        c2 = i12 * t1 + i22 * t2 + i23 * t3
        c3 = i13 * t1 + i23 * t2 + i33 * t3

        def expand(v):
            return jax.lax.dot_general(
                v, seg_t, (((1,), (0,)), ((), ())),
                preferred_element_type=jnp.float32)[:, None, :]

        r = expand(c1) * ua1[None] + expand(c2) * ua2[None] + expand(c3)
        r1 = jnp.round(r)

        diff = u - r1
        loss = contract(diff * diff)
        keep = (loss * (lsb * lsb) <= LOSS_THR).astype(jnp.float32)
        keep_e = expand(keep)

        sel = u - keep_e * diff
        o_ref[...] = sel * lsb + xl


def kernel(x):
    _, imgs, h, w = x.shape
    hp = h // WL
    wp = w // WL
    x3 = x[0]

    cols = jnp.arange(w, dtype=jnp.int32) // WL
    segs = jnp.arange(wp, dtype=jnp.int32)
    seg = (cols[:, None] == segs[None, :]).astype(jnp.float32)
    seg_t = seg.T

    out = pl.pallas_call(
        _fused_kernel,
        grid=(2 * hp,),
        in_specs=[
            pl.BlockSpec((w, wp), lambda i: (0, 0)),
            pl.BlockSpec((wp, w), lambda i: (0, 0)),
            pl.BlockSpec((imgs, WL, w), lambda i: (0, jax.lax.rem(i, hp), 0)),
        ],
        out_specs=pl.BlockSpec(
            (imgs, WL, w),
            lambda i: (0, jnp.maximum(i - hp, 0), 0)),
        out_shape=jax.ShapeDtypeStruct((imgs, h, w), jnp.float32),
        scratch_shapes=[pltpu.VMEM((8, 128), jnp.float32)],
    )(seg, seg_t, x3)

    return out[None]


# speedup vs baseline: 8.9354x; 1.0527x over previous
import jax
import jax.numpy as jnp
from jax.experimental import pallas as pl
from jax.experimental.pallas import tpu as pltpu

WL = 16
LOSS_THR = 1.0
BIT = 8


def _delta(xb, imgs, wl, w):
    xl = jnp.roll(xb, 1, axis=2)
    lane = jax.lax.broadcasted_iota(jnp.int32, (imgs, wl, w), 2)
    xl = jnp.where(lane == 0, 0.0, xl)
    return xl, xb - xl


def _fused_kernel(seg_ref, seg_t_ref, x_ref, o_ref, acc_ref):
    imgs, wl, w = x_ref.shape
    hp = pl.num_programs(0) // 2
    i = pl.program_id(0)

    @pl.when(i < hp)
    def _max_phase():
        xb = x_ref[...]
        _, xc = _delta(xb, imgs, wl, w)
        m = jnp.full((8, 128), jnp.max(jnp.abs(xc)), jnp.float32)
        prev = jnp.where(i == 0, jnp.zeros((8, 128), jnp.float32), acc_ref[...])
        acc_ref[...] = jnp.maximum(prev, m)

    @pl.when(i >= hp)
    def _fit_phase():
        in_max = jnp.max(acc_ref[...])
        lsb = 2.0 ** (jnp.round(jnp.log2(in_max / 2.0 ** (BIT - 1))) + 1.0)
        inv_lsb = 1.0 / lsb

        xb = x_ref[...]
        xl, xc = _delta(xb, imgs, wl, w)
        u = jnp.round(xc * inv_lsb)

        ua1 = u[0]
        ua2 = u[1]

        seg = seg_ref[...]
        seg_t = seg_t_ref[...]

        def contract(v):
            part = jax.lax.dot_general(
                v.reshape(imgs * wl, w), seg, (((1,), (0,)), ((), ())),
                preferred_element_type=jnp.float32)
            return jnp.sum(part.reshape(imgs, wl, w // WL), axis=1)

        t1 = contract(u * ua1[None])
        t2 = contract(u * ua2[None])
        t3 = contract(u)

        s11 = t1[0:1]
        s12 = t1[1:2]
        s22 = t2[1:2]
        s1 = t3[0:1]
        s2 = t3[1:2]
        n = jnp.float32(wl * wl)

        m11 = s22 * n - s2 * s2
        m12 = s1 * s2 - s12 * n
        m13 = s12 * s2 - s1 * s22
        det = s11 * m11 + s12 * m12 + s1 * m13
        m22 = s11 * n - s1 * s1
        m23 = s12 * s1 - s11 * s2
        m33 = s11 * s22 - s12 * s12
        det0 = det == 0.0
        rdet = 1.0 / jnp.where(det0, 1.0, det)
        i11 = jnp.where(det0, 1.0, m11 * rdet)
        i12 = jnp.where(det0, 0.0, m12 * rdet)
        i13 = jnp.where(det0, 0.0, m13 * rdet)
        i22 = jnp.where(det0, 1.0, m22 * rdet)
        i23 = jnp.where(det0, 0.0, m23 * rdet)
        i33 = jnp.where(det0, 1.0, m33 * rdet)

        c1 = i11 * t1 + i12 * t2 + i13 * t3
        c2 = i12 * t1 + i22 * t2 + i23 * t3
        c3 = i13 * t1 + i23 * t2 + i33 * t3

        def expand(v):
            vb = jnp.broadcast_to(v[:, None, :], (imgs, wl, v.shape[1]))
            return jax.lax.dot_general(
                vb.reshape(imgs * wl, v.shape[1]), seg_t,
                (((1,), (0,)), ((), ())),
                preferred_element_type=jnp.float32).reshape(imgs, wl, w)

        r = expand(c1) * ua1[None] + expand(c2) * ua2[None] + expand(c3)
        r1 = jnp.round(r)

        diff = u - r1
        loss = contract(diff * diff)
        keep = (loss * (lsb * lsb) <= LOSS_THR).astype(jnp.float32)
        keep_e = expand(keep)

        sel = u - keep_e * diff
        o_ref[...] = sel * lsb + xl


def kernel(x):
    _, imgs, h, w = x.shape
    hp = h // WL
    wp = w // WL
    x3 = x[0]

    cols = jnp.arange(w, dtype=jnp.int32) // WL
    segs = jnp.arange(wp, dtype=jnp.int32)
    seg = (cols[:, None] == segs[None, :]).astype(jnp.float32)
    seg_t = seg.T

    out = pl.pallas_call(
        _fused_kernel,
        grid=(2 * hp,),
        in_specs=[
            pl.BlockSpec((w, wp), lambda i: (0, 0)),
            pl.BlockSpec((wp, w), lambda i: (0, 0)),
            pl.BlockSpec((imgs, WL, w), lambda i: (0, jax.lax.rem(i, hp), 0)),
        ],
        out_specs=pl.BlockSpec(
            (imgs, WL, w),
            lambda i: (0, jnp.maximum(i - hp, 0), 0)),
        out_shape=jax.ShapeDtypeStruct((imgs, h, w), jnp.float32),
        scratch_shapes=[pltpu.VMEM((8, 128), jnp.float32)],
    )(seg, seg_t, x3)

    return out[None]


# speedup vs baseline: 9.9506x; 1.1136x over previous
import jax
import jax.numpy as jnp
from jax.experimental import pallas as pl
from jax.experimental.pallas import tpu as pltpu

WL = 16
LOSS_THR = 1.0
BIT = 8


def _delta(xb, imgs, wl, w):
    xl = jnp.roll(xb, 1, axis=2)
    lane = jax.lax.broadcasted_iota(jnp.int32, (imgs, wl, w), 2)
    xl = jnp.where(lane == 0, 0.0, xl)
    return xl, xb - xl


def _fused_kernel(seg_ref, seg_t_ref, x_ref, o_ref, acc_ref):
    imgs, bwl, w = x_ref.shape
    hp = pl.num_programs(0) // 2
    i = pl.program_id(0)

    @pl.when(i < hp)
    def _max_phase():
        xb = x_ref[...]
        _, xc = _delta(xb, imgs, bwl, w)
        m = jnp.full((8, 128), jnp.max(jnp.abs(xc)), jnp.float32)
        prev = jnp.where(i == 0, jnp.zeros((8, 128), jnp.float32), acc_ref[...])
        acc_ref[...] = jnp.maximum(prev, m)

    @pl.when(i >= hp)
    def _fit_phase():
        in_max = jnp.max(acc_ref[...])
        lsb = 2.0 ** (jnp.round(jnp.log2(in_max / 2.0 ** (BIT - 1))) + 1.0)
        inv_lsb = 1.0 / lsb

        xb = x_ref[...]
        xlf, xcf = _delta(xb, imgs, bwl, w)
        uf = jnp.round(xcf * inv_lsb)

        seg = seg_ref[...]
        seg_t = seg_t_ref[...]

        for s in range(bwl // WL):
            _fit_strip(uf[:, s * WL:(s + 1) * WL, :],
                       xlf[:, s * WL:(s + 1) * WL, :],
                       seg, seg_t, lsb, o_ref, s)


def _fit_strip(u, xl, seg, seg_t, lsb, o_ref, s):
        imgs, wl, w = u.shape
        ua1 = u[0]
        ua2 = u[1]

        def contract(v):
            part = jax.lax.dot_general(
                v.reshape(imgs * wl, w), seg, (((1,), (0,)), ((), ())),
                preferred_element_type=jnp.float32)
            return jnp.sum(part.reshape(imgs, wl, w // WL), axis=1)

        t1 = contract(u * ua1[None])
        t2 = contract(u * ua2[None])
        t3 = contract(u)

        s11 = t1[0:1]
        s12 = t1[1:2]
        s22 = t2[1:2]
        s1 = t3[0:1]
        s2 = t3[1:2]
        n = jnp.float32(wl * wl)

        m11 = s22 * n - s2 * s2
        m12 = s1 * s2 - s12 * n
        m13 = s12 * s2 - s1 * s22
        det = s11 * m11 + s12 * m12 + s1 * m13
        m22 = s11 * n - s1 * s1
        m23 = s12 * s1 - s11 * s2
        m33 = s11 * s22 - s12 * s12
        det0 = det == 0.0
        rdet = 1.0 / jnp.where(det0, 1.0, det)
        i11 = jnp.where(det0, 1.0, m11 * rdet)
        i12 = jnp.where(det0, 0.0, m12 * rdet)
        i13 = jnp.where(det0, 0.0, m13 * rdet)
        i22 = jnp.where(det0, 1.0, m22 * rdet)
        i23 = jnp.where(det0, 0.0, m23 * rdet)
        i33 = jnp.where(det0, 1.0, m33 * rdet)

        c1 = i11 * t1 + i12 * t2 + i13 * t3
        c2 = i12 * t1 + i22 * t2 + i23 * t3
        c3 = i13 * t1 + i23 * t2 + i33 * t3

        def expand(v):
            vb = jnp.broadcast_to(v[:, None, :], (imgs, wl, v.shape[1]))
            return jax.lax.dot_general(
                vb.reshape(imgs * wl, v.shape[1]), seg_t,
                (((1,), (0,)), ((), ())),
                preferred_element_type=jnp.float32).reshape(imgs, wl, w)

        r = expand(c1) * ua1[None] + expand(c2) * ua2[None] + expand(c3)
        r1 = jnp.round(r)

        diff = u - r1
        loss = contract(diff * diff)
        keep = (loss * (lsb * lsb) <= LOSS_THR).astype(jnp.float32)
        keep_e = expand(keep)

        sel = u - keep_e * diff
        o_ref[:, pl.ds(s * wl, wl), :] = sel * lsb + xl


def kernel(x):
    _, imgs, h, w = x.shape
    bwl = 2 * WL
    nblk = h // bwl
    wp = w // WL
    x3 = x[0]

    cols = jnp.arange(w, dtype=jnp.int32) // WL
    segs = jnp.arange(wp, dtype=jnp.int32)
    seg = (cols[:, None] == segs[None, :]).astype(jnp.float32)
    seg_t = seg.T

    out = pl.pallas_call(
        _fused_kernel,
        grid=(2 * nblk,),
        in_specs=[
            pl.BlockSpec((w, wp), lambda i: (0, 0)),
            pl.BlockSpec((wp, w), lambda i: (0, 0)),
            pl.BlockSpec((imgs, bwl, w), lambda i: (0, jax.lax.rem(i, nblk), 0)),
        ],
        out_specs=pl.BlockSpec(
            (imgs, bwl, w),
            lambda i: (0, jnp.maximum(i - nblk, 0), 0)),
        out_shape=jax.ShapeDtypeStruct((imgs, h, w), jnp.float32),
        scratch_shapes=[pltpu.VMEM((8, 128), jnp.float32)],
    )(seg, seg_t, x3)

    return out[None]
